# Initial kernel scaffold; baseline (speedup 1.0000x reference)
import jax
import jax.numpy as jnp
from jax.experimental import pallas as pl


def kernel(outputs, targets):
    raise NotImplementedError("write your pallas kernel here")


# speedup vs baseline: 16.2811x; 16.2811x over previous
import numpy as np
import jax
import jax.numpy as jnp
from jax.experimental import pallas as pl
from jax.experimental.pallas import tpu as pltpu

_NC = 100
_B = 1024
_PAD = 128
_NUM_LOSSES = _B * (_NC - 1) / 2.0


def _build_tree(num_classes):
    nodes = []

    def rec(leaves):
        if len(leaves) <= 1:
            return
        mid = len(leaves) // 2
        nodes.append((leaves[:mid], leaves[mid:]))
        rec(leaves[:mid])
        rec(leaves[mid:])

    rec(list(range(num_classes)))
    return nodes


def _build_tables():
    nodes = _build_tree(_NC)
    W = np.zeros((_PAD, 2 * _PAD), np.float32)
    Bm = np.zeros((_PAD, 2 * _PAD), np.float32)
    for i, (L, R) in enumerate(nodes):
        W[np.asarray(L), i] = 1.0 / len(L)
        W[np.asarray(R), _PAD + i] = 1.0 / len(R)
        Bm[np.asarray(L), i] = 1.0
        Bm[np.asarray(R), _PAD + i] = 1.0
    return jnp.asarray(W), jnp.asarray(Bm)


_W, _BM = _build_tables()


def _body(x_ref, t_ref, w_ref, b_ref, o_ref):
    x = x_ref[...]
    tt = t_ref[...]
    iota = jax.lax.broadcasted_iota(jnp.int32, (_B, _PAD), 1)
    onehot = (iota == tt).astype(jnp.float32)
    X1 = jnp.dot(x, w_ref[...], preferred_element_type=jnp.float32)
    X2 = jnp.dot(onehot, b_ref[...], preferred_element_type=jnp.float32)
    m0, m1 = X1[:, :_PAD], X1[:, _PAD:]
    c0, c1 = X2[:, :_PAD], X2[:, _PAD:]
    mx = jnp.maximum(m0, m1)
    lse = mx + jnp.log1p(jnp.exp(-jnp.abs(m0 - m1)))
    tot = jnp.sum((c0 + c1) * lse - c0 * m0 - c1 * m1)
    o_ref[...] = (tot * np.float32(1.0 / _NUM_LOSSES)).reshape(1, 1)


def kernel(outputs, targets):
    x = jnp.pad(outputs.astype(jnp.float32), ((0, 0), (0, _PAD - _NC)))
    t = targets.astype(jnp.int32).reshape(_B, 1)
    out = pl.pallas_call(
        _body,
        out_shape=jax.ShapeDtypeStruct((1, 1), jnp.float32),
    )(x, t, _W, _BM)
    return out[0, 0]
